# baseline (speedup 1.0000x reference)
import functools

import jax
import jax.numpy as jnp
from jax import lax
from jax.experimental import pallas as pl
from jax.experimental.pallas import tpu as pltpu
from jax.experimental.pallas import tpu_sc as plsc

N = 10000
D = 256
E = 160000
L = 16
NTILES = 32
NPT = 320
N_PAD = NTILES * NPT
CHUNK = 1600
NCHUNK = E // CHUNK
SCAN_STEPS = CHUNK // L
DC = D // L


def _sc_agg_body(xa_hbm, xp_hbm, sw_hbm, dw_hbm, sc_hbm, dc_hbm,
                 outw_hbm, outc_hbm,
                 src_buf, dst_buf, pend_src, pend_dst, row_buf, accum, sem):
    cid = lax.axis_index("c")
    sid = lax.axis_index("s")
    wid = sid * 2 + cid
    base = wid * NPT

    neg_inf = jnp.full((L,), -jnp.inf, dtype=jnp.float32)

    for x_hbm, s_hbm, d_hbm, o_hbm in (
        (xa_hbm, sw_hbm, dw_hbm, outw_hbm),
        (xp_hbm, sc_hbm, dc_hbm, outc_hbm),
    ):
        def init_row(i, carry):
            for c in range(DC):
                accum[i, pl.ds(c * L, L)] = neg_inf
            return carry
        lax.fori_loop(0, NPT + 1, init_row, 0)

        def chunk_body(k, carry):
            pltpu.sync_copy(s_hbm.at[pl.ds(k * CHUNK, CHUNK)], src_buf)
            pltpu.sync_copy(d_hbm.at[pl.ds(k * CHUNK, CHUNK)], dst_buf)

            def scan_step(s, cnt):
                sv = src_buf[pl.ds(s * L, L)]
                dv = dst_buf[pl.ds(s * L, L)]
                dl = dv - base
                m = (dl >= 0) & (dl < NPT)
                cum = plsc.cumsum(m.astype(jnp.int32))
                pos = cnt + cum - 1
                plsc.store_scatter(pend_src, [pos], sv, mask=m)
                plsc.store_scatter(pend_dst, [pos], dl, mask=m)
                return cnt + cum[L - 1]

            cnt = lax.fori_loop(0, SCAN_STEPS, scan_step, jnp.int32(0))
            pend_src[pl.ds(cnt, L)] = jnp.zeros((L,), jnp.int32)
            pend_dst[pl.ds(cnt, L)] = jnp.full((L,), NPT, jnp.int32)
            nb = (cnt + L - 1) // L

            def gather_body(g, carry):
                pltpu.async_copy(
                    x_hbm.at[pend_src.at[pl.ds(g * L, L)]], row_buf, sem
                ).wait()
                dvec = pend_dst[pl.ds(g * L, L)]
                for j in range(L):
                    d = dvec[j]
                    for c in range(DC):
                        sl = pl.ds(c * L, L)
                        accum[d, sl] = jnp.maximum(accum[d, sl], row_buf[j, sl])
                return carry
            lax.fori_loop(0, nb, gather_body, 0)
            return carry
        lax.fori_loop(0, NCHUNK, chunk_body, 0)

        def fin_row(i, carry):
            for c in range(DC):
                sl = pl.ds(c * L, L)
                v = accum[i, sl]
                accum[i, sl] = jnp.where(v == neg_inf, 0.0, v)
            return carry
        lax.fori_loop(0, NPT, fin_row, 0)
        pltpu.sync_copy(accum.at[pl.ds(0, NPT)], o_hbm.at[pl.ds(base, NPT)])


def _sc_aggregate(xa, xp, src_w, dst_w, src_c, dst_c):
    mesh = plsc.VectorSubcoreMesh(core_axis_name="c", subcore_axis_name="s")
    return pl.kernel(
        _sc_agg_body,
        out_type=[jax.ShapeDtypeStruct((N_PAD, D), jnp.float32)] * 2,
        mesh=mesh,
        scratch_types=[
            pltpu.VMEM((CHUNK,), jnp.int32),
            pltpu.VMEM((CHUNK,), jnp.int32),
            pltpu.VMEM((CHUNK + L,), jnp.int32),
            pltpu.VMEM((CHUNK + L,), jnp.int32),
            pltpu.VMEM((L, D), jnp.float32),
            pltpu.VMEM((NPT + 1, D), jnp.float32),
            pltpu.SemaphoreType.DMA,
        ],
        compiler_params=pltpu.CompilerParams(needs_layout_passes=False),
    )(xa, xp, src_w, dst_w, src_c, dst_c)


def _mm_body(xa_ref, xp_ref, aw_ref, ac_ref, wra_ref, bra_ref, wrp_ref,
             brp_ref, ww_ref, wc_ref, oa_ref, op_ref):
    dn = (((1,), (1,)), ((), ()))
    oa_ref[...] = lax.dot_general(
        xa_ref[...], wra_ref[...], dn, preferred_element_type=jnp.float32
    ) + bra_ref[...]
    op_ref[...] = (
        lax.dot_general(xp_ref[...], wrp_ref[...], dn,
                        preferred_element_type=jnp.float32)
        + brp_ref[...]
        + lax.dot_general(aw_ref[...], ww_ref[...], dn,
                          preferred_element_type=jnp.float32)
        + lax.dot_general(ac_ref[...], wc_ref[...], dn,
                          preferred_element_type=jnp.float32)
    )


def _tc_matmuls(xa, xp, agg_w, agg_c, wra, bra, wrp, brp, ww, wc):
    bm = 1000
    grid = (N // bm,)
    row_spec = pl.BlockSpec((bm, D), lambda i: (i, 0))
    w_spec = pl.BlockSpec((D, D), lambda i: (0, 0))
    b_spec = pl.BlockSpec((1, D), lambda i: (0, 0))
    return pl.pallas_call(
        _mm_body,
        grid=grid,
        in_specs=[row_spec, row_spec, row_spec, row_spec,
                  w_spec, b_spec, w_spec, b_spec, w_spec, w_spec],
        out_specs=[row_spec, row_spec],
        out_shape=[jax.ShapeDtypeStruct((N, D), jnp.float32)] * 2,
    )(xa, xp, agg_w, agg_c, wra, bra.reshape(1, D), wrp, brp.reshape(1, D),
      ww, wc)


@jax.jit
def kernel(x_author, x_paper, edge_index_writes, edge_index_cites,
           W_writes, W_cites, W_root_author, b_root_author,
           W_root_paper, b_root_paper):
    agg_w_pad, agg_c_pad = _sc_aggregate(
        x_author, x_paper,
        edge_index_writes[0], edge_index_writes[1],
        edge_index_cites[0], edge_index_cites[1],
    )
    out_author, out_paper = _tc_matmuls(
        x_author, x_paper, agg_w_pad[:N], agg_c_pad[:N],
        W_root_author, b_root_author, W_root_paper, b_root_paper,
        W_writes, W_cites,
    )
    return (out_author, out_paper)


# speedup vs baseline: 1.1540x; 1.1540x over previous
import functools

import jax
import jax.numpy as jnp
from jax import lax
from jax.experimental import pallas as pl
from jax.experimental.pallas import tpu as pltpu
from jax.experimental.pallas import tpu_sc as plsc

N = 10000
D = 256
E = 160000
L = 16
NTILES = 32
NPT = 320
N_PAD = NTILES * NPT
CHUNK = 3200
NCHUNK = E // CHUNK
SCAN_STEPS = CHUNK // L
CAP = CHUNK // L
DC = D // L
GB = 16


def _sc_agg_body(x_hbm, s_hbm, d_hbm, out_hbm,
                 es0, ed0, es1, ed1, pend_src, pend_dst,
                 msrc, mdst, row0, row1, accum,
                 sem_e0, sem_e1, sem_g0, sem_g1):
    cid = lax.axis_index("c")
    sid = lax.axis_index("s")
    wid = sid * 2 + cid
    base = wid * NPT

    neg_inf = jnp.full((L,), -jnp.inf, dtype=jnp.float32)
    lane_base = jnp.arange(L, dtype=jnp.int32) * CAP
    ebufs = ((es0, ed0, sem_e0), (es1, ed1, sem_e1))
    rbufs = ((row0, sem_g0), (row1, sem_g1))

    def rel_body(r, carry0):
        ebase = r * E

        def init_row(i, c2):
            for c in range(DC):
                accum[i, pl.ds(c * L, L)] = neg_inf
            return c2
        lax.fori_loop(0, NPT + 1, init_row, 0)

        pltpu.async_copy(s_hbm.at[pl.ds(ebase, CHUNK)], es0, sem_e0)
        pltpu.async_copy(d_hbm.at[pl.ds(ebase, CHUNK)], ed0, sem_e0)

        def chunk_pair(k2, c2):
            for b in range(2):
                eb_s, eb_d, eb_sem = ebufs[b]
                k = k2 * 2 + b
                off = ebase + k * CHUNK
                pltpu.make_async_copy(
                    s_hbm.at[pl.ds(off, CHUNK)], eb_s, eb_sem).wait()
                pltpu.make_async_copy(
                    d_hbm.at[pl.ds(off, CHUNK)], eb_d, eb_sem).wait()
                nb_s, nb_d, nb_sem = ebufs[1 - b]

                @pl.when(k + 1 < NCHUNK)
                def _():
                    noff = ebase + (k + 1) * CHUNK
                    pltpu.async_copy(s_hbm.at[pl.ds(noff, CHUNK)], nb_s, nb_sem)
                    pltpu.async_copy(d_hbm.at[pl.ds(noff, CHUNK)], nb_d, nb_sem)

                def scan_step(s, cnt_vec):
                    sv = eb_s[pl.ds(s * L, L)]
                    dv = eb_d[pl.ds(s * L, L)]
                    dl = dv - base
                    m = (dl >= 0) & (dl < NPT)
                    pos = lane_base + cnt_vec
                    plsc.store_scatter(pend_src, [pos], sv + r * N, mask=m)
                    plsc.store_scatter(pend_dst, [pos], dl, mask=m)
                    return cnt_vec + m.astype(jnp.int32)

                cnt_vec = lax.fori_loop(
                    0, SCAN_STEPS, scan_step,
                    jnp.zeros((L,), jnp.int32))

                o = jnp.int32(0)
                for lane in range(L):
                    nl = cnt_vec[lane]

                    def copy_body(i, o_in):
                        v = pend_src[pl.ds(lane * CAP + i * L, L)]
                        msrc[pl.ds(o_in + i * L, L)] = v
                        w = pend_dst[pl.ds(lane * CAP + i * L, L)]
                        mdst[pl.ds(o_in + i * L, L)] = w
                        return o_in
                    lax.fori_loop(0, (nl + L - 1) // L,
                                  functools.partial(copy_body), o)
                    o = o + nl
                msrc[pl.ds(o, L)] = jnp.full((L,), r * N, jnp.int32)
                mdst[pl.ds(o, L)] = jnp.full((L,), NPT, jnp.int32)
                nb = (o + L - 1) // L

                @pl.when(nb > 0)
                def _():
                    pltpu.async_copy(
                        x_hbm.at[msrc.at[pl.ds(0, GB)]], row0, sem_g0)

                def gather_pair(g2, c3):
                    for gb in range(2):
                        rbuf, rsem = rbufs[gb]
                        orbuf, orsem = rbufs[1 - gb]
                        g = g2 * 2 + gb

                        @pl.when(g < nb)
                        def _():
                            pltpu.make_async_copy(
                                x_hbm.at[msrc.at[pl.ds(g * GB, GB)]],
                                rbuf, rsem).wait()

                            @pl.when(g + 1 < nb)
                            def _():
                                pltpu.async_copy(
                                    x_hbm.at[msrc.at[pl.ds((g + 1) * GB, GB)]],
                                    orbuf, orsem)

                            dvec = mdst[pl.ds(g * GB, GB)]
                            for j in range(GB):
                                d = dvec[j]
                                for c in range(DC):
                                    sl = pl.ds(c * L, L)
                                    accum[d, sl] = jnp.maximum(
                                        accum[d, sl], rbuf[j, sl])
                    return c3
                lax.fori_loop(0, (nb + 1) // 2, gather_pair, 0)
            return c2
        lax.fori_loop(0, NCHUNK // 2, chunk_pair, 0)

        def fin_row(i, c2):
            for c in range(DC):
                sl = pl.ds(c * L, L)
                v = accum[i, sl]
                accum[i, sl] = jnp.where(v == neg_inf, 0.0, v)
            return c2
        lax.fori_loop(0, NPT, fin_row, 0)
        pltpu.sync_copy(accum.at[pl.ds(0, NPT)],
                        out_hbm.at[r, pl.ds(base, NPT)])
        return carry0
    lax.fori_loop(0, 2, rel_body, 0)


def _sc_aggregate(x2, src2, dst2):
    mesh = plsc.VectorSubcoreMesh(core_axis_name="c", subcore_axis_name="s")
    return pl.kernel(
        _sc_agg_body,
        out_type=jax.ShapeDtypeStruct((2, N_PAD, D), jnp.float32),
        mesh=mesh,
        scratch_types=[
            pltpu.VMEM((CHUNK,), jnp.int32),
            pltpu.VMEM((CHUNK,), jnp.int32),
            pltpu.VMEM((CHUNK,), jnp.int32),
            pltpu.VMEM((CHUNK,), jnp.int32),
            pltpu.VMEM((CHUNK,), jnp.int32),
            pltpu.VMEM((CHUNK,), jnp.int32),
            pltpu.VMEM((CHUNK + L,), jnp.int32),
            pltpu.VMEM((CHUNK + L,), jnp.int32),
            pltpu.VMEM((GB, D), jnp.float32),
            pltpu.VMEM((GB, D), jnp.float32),
            pltpu.VMEM((NPT + 1, D), jnp.float32),
            pltpu.SemaphoreType.DMA,
            pltpu.SemaphoreType.DMA,
            pltpu.SemaphoreType.DMA,
            pltpu.SemaphoreType.DMA,
        ],
        compiler_params=pltpu.CompilerParams(needs_layout_passes=False),
    )(x2, src2, dst2)


def _mm_body(xa_ref, xp_ref, aw_ref, ac_ref, wra_ref, bra_ref, wrp_ref,
             brp_ref, ww_ref, wc_ref, oa_ref, op_ref):
    dn = (((1,), (1,)), ((), ()))
    oa_ref[...] = lax.dot_general(
        xa_ref[...], wra_ref[...], dn, preferred_element_type=jnp.float32
    ) + bra_ref[...]
    op_ref[...] = (
        lax.dot_general(xp_ref[...], wrp_ref[...], dn,
                        preferred_element_type=jnp.float32)
        + brp_ref[...]
        + lax.dot_general(aw_ref[...], ww_ref[...], dn,
                          preferred_element_type=jnp.float32)
        + lax.dot_general(ac_ref[...], wc_ref[...], dn,
                          preferred_element_type=jnp.float32)
    )


def _tc_matmuls(xa, xp, agg_w, agg_c, wra, bra, wrp, brp, ww, wc):
    bm = 1000
    grid = (N // bm,)
    row_spec = pl.BlockSpec((bm, D), lambda i: (i, 0))
    w_spec = pl.BlockSpec((D, D), lambda i: (0, 0))
    b_spec = pl.BlockSpec((1, D), lambda i: (0, 0))
    return pl.pallas_call(
        _mm_body,
        grid=grid,
        in_specs=[row_spec, row_spec, row_spec, row_spec,
                  w_spec, b_spec, w_spec, b_spec, w_spec, w_spec],
        out_specs=[row_spec, row_spec],
        out_shape=[jax.ShapeDtypeStruct((N, D), jnp.float32)] * 2,
    )(xa, xp, agg_w, agg_c, wra, bra.reshape(1, D), wrp, brp.reshape(1, D),
      ww, wc)


@jax.jit
def kernel(x_author, x_paper, edge_index_writes, edge_index_cites,
           W_writes, W_cites, W_root_author, b_root_author,
           W_root_paper, b_root_paper):
    x2 = jnp.concatenate([x_author, x_paper], axis=0)
    src2 = jnp.concatenate([edge_index_writes[0], edge_index_cites[0]])
    dst2 = jnp.concatenate([edge_index_writes[1], edge_index_cites[1]])
    agg = _sc_aggregate(x2, src2, dst2)
    out_author, out_paper = _tc_matmuls(
        x_author, x_paper, agg[0, :N], agg[1, :N],
        W_root_author, b_root_author, W_root_paper, b_root_paper,
        W_writes, W_cites,
    )
    return (out_author, out_paper)
